# Initial kernel scaffold; baseline (speedup 1.0000x reference)
import jax
import jax.numpy as jnp
from jax.experimental import pallas as pl


def kernel(x, token_table, pos_table):
    raise NotImplementedError("write your pallas kernel here")


# speedup vs baseline: 1.4279x; 1.4279x over previous
import functools

import jax
import jax.numpy as jnp
from jax import lax
from jax.experimental import pallas as pl
from jax.experimental.pallas import tpu as pltpu
from jax.experimental.pallas import tpu_sc as plsc

B = 4096
L = 200
D = 32
N = B * L
NC = 2
NS = 16
NW = NC * NS
PER_W = N // NW
C = 1600
NCHUNKS = PER_W // C
RB = C // L
SUB = 80
K = C // SUB
LANES = 16


def _body(x_hbm, tok_hbm, pos_hbm, out_hbm, idx_v, rows_v, pos_v, sem_g):
    wid = lax.axis_index("s") * NC + lax.axis_index("c")
    pltpu.sync_copy(pos_hbm, pos_v)

    @pl.loop(0, NCHUNKS)
    def _chunk(c):
        g = wid * NCHUNKS + c

        pltpu.sync_copy(x_hbm.at[g], idx_v)

        @pl.loop(0, K)
        def _fire(j):
            pltpu.async_copy(
                tok_hbm.at[idx_v.at[j]],
                rows_v.at[pl.ds(j * SUB, SUB)],
                sem_g,
            )

        pltpu.make_async_copy(tok_hbm.at[pl.ds(0, C)], rows_v, sem_g).wait()

        @pl.loop(0, L)
        def _mul(l):
            p0 = pos_v[l, pl.ds(0, LANES)]
            p1 = pos_v[l, pl.ds(LANES, LANES)]
            for r in range(RB):
                row = r * L + l
                rows_v[row, pl.ds(0, LANES)] = rows_v[row, pl.ds(0, LANES)] * p0
                rows_v[row, pl.ds(LANES, LANES)] = (
                    rows_v[row, pl.ds(LANES, LANES)] * p1
                )

        pltpu.sync_copy(rows_v, out_hbm.at[pl.ds(g * C, C)])


@jax.jit
def _encode(x3, token_table, pos_table):
    mesh = plsc.VectorSubcoreMesh(core_axis_name="c", subcore_axis_name="s")
    k = pl.kernel(
        _body,
        out_type=jax.ShapeDtypeStruct((N, D), jnp.float32),
        mesh=mesh,
        compiler_params=pltpu.CompilerParams(use_tc_tiling_on_sc=False),
        scratch_types=[
            pltpu.VMEM((K, SUB), jnp.int32),
            pltpu.VMEM((C, D), jnp.float32),
            pltpu.VMEM((L, D), jnp.float32),
            pltpu.SemaphoreType.DMA,
        ],
    )
    return k(x3, token_table, pos_table)


def kernel(x, token_table, pos_table):
    x3 = x.astype(jnp.int32).reshape(N // C, K, SUB)
    out = _encode(x3, token_table, pos_table)
    return out.reshape(B, L, D)
